# baseline (speedup 1.0000x reference)
import functools

import jax
import jax.numpy as jnp
from jax import lax
from jax.experimental import pallas as pl
from jax.experimental.pallas import tpu as pltpu
from jax.experimental.pallas import tpu_sc as plsc

_N_TOK = 32768
_ENT_W = 0.001
_EPS = 0.0001
_A_CLIP = 9.210240366975849
_H_CLIP = 0.0010210290545737

_NC = 1
_NS = 16
_NW = _NC * _NS
_L = 16
_CHUNK = _N_TOK // _NW
_NVEC = _CHUNK // _L


def _entropy_vec(x):
    a = jnp.abs(x)
    z = jnp.exp(-a)
    t1 = 1.0 + z
    t2 = 2.0 + z
    r = 1.0 / (t1 * t2)
    u = z * t1 * r
    w = z * t2 * r
    u2 = u * u
    log1pz = 2.0 * u * (1.0 + u2 * (1.0 / 3.0 + u2 * (1.0 / 5.0)))
    h = log1pz + a * w
    return jnp.where(a > _A_CLIP, _H_CLIP, h)


@functools.partial(
    pl.kernel,
    out_type=jax.ShapeDtypeStruct((_NW, _L), jnp.float32),
    mesh=plsc.VectorSubcoreMesh(core_axis_name="c", subcore_axis_name="s", num_cores=1),
    compiler_params=pltpu.CompilerParams(needs_layout_passes=False),
    scratch_types=[
        pltpu.VMEM((_CHUNK,), jnp.float32),
        pltpu.VMEM((_L,), jnp.float32),
    ],
)
def _entropy_loss_kernel(x_hbm, out_hbm, xv, stage_v):
    cid = lax.axis_index("c")
    sid = lax.axis_index("s")
    wid = cid * _NS + sid

    pltpu.sync_copy(x_hbm.at[pl.ds(wid * _CHUNK, _CHUNK)], xv)

    def body(i, accs):
        a0, a1 = accs
        return (
            a0 + _entropy_vec(xv[pl.ds(i * (2 * _L), _L)]),
            a1 + _entropy_vec(xv[pl.ds(i * (2 * _L) + _L, _L)]),
        )

    zero = jnp.zeros((_L,), jnp.float32)
    a0, a1 = lax.fori_loop(0, _NVEC // 2, body, (zero, zero))
    acc = a0 + a1

    partial = jnp.sum(acc) * (-_ENT_W / _N_TOK)
    stage_v[...] = jnp.full((_L,), partial, jnp.float32)
    pltpu.sync_copy(stage_v, out_hbm.at[wid])


def kernel(logits, targets):
    del targets
    out = _entropy_loss_kernel(logits.reshape(_N_TOK))
    return jnp.sum(out[:, 0])


# speedup vs baseline: 1.0024x; 1.0024x over previous
import functools

import jax
import jax.numpy as jnp
from jax import lax
from jax.experimental import pallas as pl
from jax.experimental.pallas import tpu as pltpu
from jax.experimental.pallas import tpu_sc as plsc

_N_TOK = 32768
_ENT_W = 0.001
_EPS = 0.0001
_A_CLIP = 9.210240366975849
_H_CLIP = 0.0010210290545737

_NC = 1
_NS = 16
_NW = _NC * _NS
_L = 16
_CHUNK = _N_TOK // _NW
_NVEC = _CHUNK // _L


def _entropy_vec(x):
    a = jnp.abs(x)
    z = jnp.exp(-a)
    t1 = 1.0 + z
    t2 = 2.0 + z
    r = 1.0 / (t1 * t2)
    u = z * t1 * r
    w = z * t2 * r
    u2 = u * u
    log1pz = 2.0 * u * (1.0 + u2 * (1.0 / 3.0 + u2 * (1.0 / 5.0)))
    h = log1pz + a * w
    return jnp.where(a > _A_CLIP, _H_CLIP, h)


@functools.partial(
    pl.kernel,
    out_type=jax.ShapeDtypeStruct((_NW, _L), jnp.float32),
    mesh=plsc.VectorSubcoreMesh(core_axis_name="c", subcore_axis_name="s", num_cores=1),
    compiler_params=pltpu.CompilerParams(needs_layout_passes=False),
    scratch_types=[
        pltpu.VMEM((_CHUNK,), jnp.float32),
        pltpu.VMEM((_L,), jnp.float32),
        pltpu.SemaphoreType.DMA,
        pltpu.SemaphoreType.DMA,
    ],
)
def _entropy_loss_kernel(x_hbm, out_hbm, xv, stage_v, sem_a, sem_b):
    cid = lax.axis_index("c")
    sid = lax.axis_index("s")
    wid = cid * _NS + sid

    _HALF = _CHUNK // 2
    cp_a = pltpu.make_async_copy(
        x_hbm.at[pl.ds(wid * _CHUNK, _HALF)], xv.at[pl.ds(0, _HALF)], sem_a
    )
    cp_b = pltpu.make_async_copy(
        x_hbm.at[pl.ds(wid * _CHUNK + _HALF, _HALF)], xv.at[pl.ds(_HALF, _HALF)], sem_b
    )
    cp_a.start()
    cp_b.start()

    def body(i, accs):
        a0, a1 = accs
        return (
            a0 + _entropy_vec(xv[pl.ds(i * (2 * _L), _L)]),
            a1 + _entropy_vec(xv[pl.ds(i * (2 * _L) + _L, _L)]),
        )

    zero = jnp.zeros((_L,), jnp.float32)
    cp_a.wait()
    accs = lax.fori_loop(0, _NVEC // 4, body, (zero, zero))
    cp_b.wait()
    a0, a1 = lax.fori_loop(_NVEC // 4, _NVEC // 2, body, accs)
    acc = a0 + a1

    partial = jnp.sum(acc) * (-_ENT_W / _N_TOK)
    stage_v[...] = jnp.full((_L,), partial, jnp.float32)
    pltpu.sync_copy(stage_v, out_hbm.at[wid])


def kernel(logits, targets):
    del targets
    out = _entropy_loss_kernel(logits.reshape(_N_TOK))
    return jnp.sum(out[:, 0])
